# Initial kernel scaffold; baseline (speedup 1.0000x reference)
import jax
import jax.numpy as jnp
from jax.experimental import pallas as pl


def kernel(q, k, q_weight, k_weight):
    raise NotImplementedError("write your pallas kernel here")


# speedup vs baseline: 1.4728x; 1.4728x over previous
import jax
import jax.numpy as jnp
from jax.experimental import pallas as pl
from jax.experimental.pallas import tpu as pltpu

_EPS = 1e-6
_B, _D1, _D2 = 16384, 6144, 1024
_ROWS = 256


def _body(q_ref, k_ref, qw_ref, kw_ref, oq_ref, ok_ref):
    qf = q_ref[...]
    q_inv = jax.lax.rsqrt(
        jnp.sum(qf * qf, axis=1, keepdims=True) * (1.0 / _D1) + _EPS
    )
    oq_ref[...] = qf * q_inv * qw_ref[...]

    kf = k_ref[...]
    k_inv = jax.lax.rsqrt(
        jnp.sum(kf * kf, axis=1, keepdims=True) * (1.0 / _D2) + _EPS
    )
    ok_ref[...] = kf * k_inv * kw_ref[...]


def kernel(q, k, q_weight, k_weight):
    grid = (_B // _ROWS,)
    qw = q_weight.reshape(1, _D1)
    kw = k_weight.reshape(1, _D2)
    out_q, out_k = pl.pallas_call(
        _body,
        grid=grid,
        in_specs=[
            pl.BlockSpec((_ROWS, _D1), lambda i: (i, 0)),
            pl.BlockSpec((_ROWS, _D2), lambda i: (i, 0)),
            pl.BlockSpec((1, _D1), lambda i: (0, 0)),
            pl.BlockSpec((1, _D2), lambda i: (0, 0)),
        ],
        out_specs=[
            pl.BlockSpec((_ROWS, _D1), lambda i: (i, 0)),
            pl.BlockSpec((_ROWS, _D2), lambda i: (i, 0)),
        ],
        out_shape=[
            jax.ShapeDtypeStruct((_B, _D1), q.dtype),
            jax.ShapeDtypeStruct((_B, _D2), k.dtype),
        ],
        compiler_params=pltpu.CompilerParams(
            dimension_semantics=("parallel",),
            vmem_limit_bytes=56 * 1024 * 1024,
        ),
        name="fused_qk_rmsnorm",
    )(q, k, qw, kw)
    return out_q, out_k
